# Initial kernel scaffold; baseline (speedup 1.0000x reference)
import jax
import jax.numpy as jnp
from jax.experimental import pallas as pl


def kernel(log_w, particles, observation, A, C, log_sigma_x, log_sigma_y):
    raise NotImplementedError("write your pallas kernel here")


# speedup vs baseline: 2.9305x; 2.9305x over previous
import functools

import jax
import jax.numpy as jnp
import numpy as np
from jax import lax
from jax.experimental import pallas as pl
from jax.experimental.pallas import tpu as pltpu
from jax.experimental.pallas import tpu_sc as plsc

N = 1048576
D = 16
LOG2PI = float(np.log(2.0 * np.pi))


_NC, _NS = 2, 16
_NW = _NC * _NS
_ROWS_PER_W = N // _NW
_CHUNK = 2048
_GWIN = 128


def _sc_gather_kernel(table_hbm, idx_hbm, out_hbm, idx_v, rows_v, sem, gsem):
    wid = lax.axis_index("s") * _NC + lax.axis_index("c")
    base = wid * _ROWS_PER_W

    @pl.loop(0, _ROWS_PER_W, step=_CHUNK)
    def _(off):
        start = base + off
        pltpu.sync_copy(idx_hbm.at[pl.ds(start, _CHUNK)], idx_v)

        @pl.loop(0, _CHUNK, step=_GWIN)
        def _(c):
            pltpu.async_copy(
                table_hbm.at[idx_v.at[pl.ds(c, _GWIN)]],
                rows_v.at[pl.ds(c, _GWIN)],
                gsem,
            )

        pltpu.make_async_copy(
            table_hbm.at[idx_v.at[pl.ds(0, _GWIN)]],
            rows_v.at[pl.ds(0, _GWIN)],
            gsem,
        ).wait()

        @pl.loop(_GWIN, _CHUNK, step=_GWIN)
        def _(c):
            pltpu.make_async_copy(
                table_hbm.at[idx_v.at[pl.ds(c, _GWIN)]],
                rows_v.at[pl.ds(c, _GWIN)],
                gsem,
            ).wait()

        pltpu.sync_copy(rows_v, out_hbm.at[pl.ds(start, _CHUNK)])


def _sc_gather(particles, idx):
    mesh = plsc.VectorSubcoreMesh(core_axis_name="c", subcore_axis_name="s")
    k = pl.kernel(
        _sc_gather_kernel,
        out_type=jax.ShapeDtypeStruct((N, D), jnp.float32),
        mesh=mesh,
        scratch_types=[
            pltpu.VMEM((_CHUNK,), jnp.int32),
            pltpu.VMEM((_CHUNK, D), jnp.float32),
            pltpu.SemaphoreType.DMA,
            pltpu.SemaphoreType.DMA,
        ],
        compiler_params=pltpu.CompilerParams(use_tc_tiling_on_sc=False),
    )
    return k(particles, idx)


_RB = 8192
_NBLK = N // _RB


def _tc_math_kernel(pr_ref, noise_ref, logw_ref, at_ref, ct_ref, obs_ref,
                    lsx_ref, lsy_ref, scal_ref,
                    lw_out, next_out, inc_out):
    x = pr_ref[...]
    noise = noise_ref[...]
    a_t = at_ref[...]
    c_t = ct_ref[...]
    obs = obs_ref[...]
    lsx = lsx_ref[...]
    lsy = lsy_ref[...]
    flag = scal_ref[0, 0]

    mu = jnp.dot(x, a_t, preferred_element_type=jnp.float32)
    sig_p_log = lsx + jnp.float32(np.log(1.1))
    nxt = mu + noise * jnp.exp(sig_p_log)
    d = nxt - mu

    inv_sx = jnp.exp(-lsx)
    inv_sp = jnp.exp(-sig_p_log)
    trans = jnp.sum(-0.5 * (d * inv_sx) ** 2 - lsx - 0.5 * LOG2PI, axis=-1)
    prop = jnp.sum(-0.5 * (d * inv_sp) ** 2 - sig_p_log - 0.5 * LOG2PI,
                   axis=-1)
    emis_mu = jnp.dot(nxt, c_t, preferred_element_type=jnp.float32)
    e = (obs - emis_mu) * jnp.exp(-lsy)
    emis = jnp.sum(-0.5 * e * e - lsy - 0.5 * LOG2PI, axis=-1)
    inc = trans + emis - prop

    lw_r = flag * jnp.float32(-np.log(float(N))) + (1.0 - flag) * logw_ref[0, 0, :]
    lw_out[0, 0, :] = lw_r + inc
    inc_out[0, 0, :] = inc
    next_out[...] = nxt


def _tc_math(particles_r, noise, log_w, A, C, log_sigma_x, log_sigma_y,
             observation, flag):
    logw3 = log_w.reshape(_NBLK, 1, _RB)
    row_spec = pl.BlockSpec((_RB, D), lambda i: (i, 0))
    vec_spec = pl.BlockSpec((1, 1, _RB), lambda i: (i, 0, 0))
    full2 = pl.BlockSpec((D, D), lambda i: (0, 0))
    full1 = pl.BlockSpec((1, D), lambda i: (0, 0))
    out = pl.pallas_call(
        _tc_math_kernel,
        grid=(_NBLK,),
        in_specs=[row_spec, row_spec, vec_spec, full2, full2, full1, full1,
                  full1, pl.BlockSpec((1, 1), lambda i: (0, 0))],
        out_specs=[vec_spec, row_spec, vec_spec],
        out_shape=[
            jax.ShapeDtypeStruct((_NBLK, 1, _RB), jnp.float32),
            jax.ShapeDtypeStruct((N, D), jnp.float32),
            jax.ShapeDtypeStruct((_NBLK, 1, _RB), jnp.float32),
        ],
    )(particles_r, noise, logw3, A.T, C.T, observation.reshape(1, D),
      log_sigma_x.reshape(1, D), log_sigma_y.reshape(1, D),
      flag.reshape(1, 1))
    return out[0].reshape(N), out[1], out[2].reshape(N)


def kernel(log_w, particles, observation, A, C, log_sigma_x, log_sigma_y):
    n = log_w.shape[0]
    step_key = jax.random.key(42)
    resample_key, proposal_key = jax.random.split(step_key)

    w = jax.nn.softmax(log_w)
    ess_e = 1.0 / (jnp.sum(w * w) * n)
    cum = jnp.cumsum(w)
    u0 = jax.random.uniform(resample_key, ())
    u = (u0 + jnp.arange(n, dtype=jnp.float32)) / n
    idx = jnp.clip(jnp.searchsorted(cum, u), 0, n - 1)
    do_resample = ess_e < 0.5
    ancestor_ix = jnp.where(do_resample, idx, jnp.arange(n))

    noise = jax.random.normal(proposal_key, particles.shape, particles.dtype)

    particles_r = _sc_gather(particles, ancestor_ix)
    flag = do_resample.astype(jnp.float32)
    log_w_new, next_particles, inc_weight = _tc_math(
        particles_r, noise, log_w, A, C, log_sigma_x, log_sigma_y,
        observation, flag)

    return (log_w_new, next_particles, ess_e, ancestor_ix, inc_weight)


# speedup vs baseline: 12.0597x; 4.1153x over previous
import functools

import jax
import jax.numpy as jnp
import numpy as np
from jax import lax
from jax.experimental import pallas as pl
from jax.experimental.pallas import tpu as pltpu
from jax.experimental.pallas import tpu_sc as plsc

N = 1048576
D = 16
LOG2PI = float(np.log(2.0 * np.pi))

_NC, _NS = 2, 16
_NW = _NC * _NS
_C = N // _NW
_CS = 1024
_CHK = 2048
_GWIN = 128
_LANE = 16


def _lanes():
    return lax.iota(jnp.int32, _LANE)


def _a_of(cumv, jv, u0v, flagv):
    T = cumv * jnp.float32(N)
    y = T - u0v
    i0 = y.astype(jnp.int32)
    base = i0 - 2
    cnt = jnp.zeros((_LANE,), jnp.int32)
    for m in range(6):
        k = base + m
        cond = ((u0v + k.astype(jnp.float32)) <= T) | (k < 0)
        cnt = cnt + cond.astype(jnp.int32)
    a = jnp.clip(base + cnt, 0, N)
    return jnp.where(flagv, a, jv + 1)


def _shift1(v, carry):
    lanes = _lanes()
    rot = v.at[(lanes + (_LANE - 1)) & (_LANE - 1)].get(
        mode="promise_in_bounds")
    return jnp.where(lanes == 0, carry, rot)


def _sc_resample_kernel(cum_hbm, cs_hbm, scal_hbm, parts_hbm,
                        anc_hbm, rows_hbm,
                        csbuf, cumbuf, mbuf, rowsbuf, scalbuf, gsem):
    wid = lax.axis_index("s") * _NC + lax.axis_index("c")
    s = wid * _C

    pltpu.sync_copy(cs_hbm, csbuf)
    pltpu.sync_copy(scal_hbm, scalbuf)
    u0v = scalbuf[0, :]
    flagv = scalbuf[1, :] > jnp.float32(0.5)

    def ph1(k, st):
        cnt, smax = st
        cv = csbuf[pl.ds(k * _LANE, _LANE)]
        jv = (_lanes() + k * _LANE) * _CS + (_CS - 1)
        sv = _a_of(cv, jv, u0v, flagv)
        le = sv <= s
        cnt = cnt + jnp.sum(le.astype(jnp.int32))
        smax = jnp.maximum(smax, jnp.max(jnp.where(le, sv, 0)))
        return cnt, smax

    kk, carry0 = lax.fori_loop(0, _CS // _LANE, ph1,
                               (jnp.int32(0), jnp.int32(0)))
    t0 = kk * _CS

    zer = jnp.zeros((_LANE,), jnp.int32)

    @pl.loop(0, _C, step=_LANE)
    def _(i):
        mbuf[pl.ds(i, _LANE)] = zer

    def cond(st):
        return jnp.logical_not(st[2])

    def body(st):
        t, carry_a, _ = st
        tt = pl.multiple_of(jnp.minimum(t, N - _CHK), 8)
        pltpu.sync_copy(cum_hbm.at[pl.ds(tt, _CHK)], cumbuf)

        def inner(k, carry_a):
            cv = cumbuf[pl.ds(k * _LANE, _LANE)]
            jv = _lanes() + (tt + k * _LANE)
            av = _a_of(cv, jv, u0v, flagv)
            aprev = _shift1(av, carry_a)
            pos = jnp.maximum(aprev - s, 0)
            mask = (av > aprev) & (av > s) & (pos < _C)
            plsc.store_scatter(mbuf, [pos], jv + 1, mask=mask)
            return jnp.max(av)

        carry_a = lax.fori_loop(0, _CHK // _LANE, inner, carry_a)
        t2 = tt + _CHK
        return t2, carry_a, (carry_a >= s + _C) | (t2 >= N)

    lax.while_loop(cond, body, (t0, carry0, t0 >= N))

    pltpu.sync_copy(cum_hbm.at[pl.ds(N - _LANE, _LANE)],
                    cumbuf.at[pl.ds(0, _LANE)])
    cv = cumbuf[pl.ds(0, _LANE)]
    av = _a_of(cv, _lanes() + (N - _LANE), u0v, flagv)
    a_last = jnp.max(av)
    pos_t = jnp.maximum(a_last - s, 0)
    maskt = (_lanes() == 0) & ((a_last - s) < _C) & (a_last <= N - 1)
    plsc.store_scatter(mbuf, [jnp.zeros((_LANE,), jnp.int32) + pos_t],
                       jnp.zeros((_LANE,), jnp.int32) + N, mask=maskt)

    def ph4(k, carry):
        v = mbuf[pl.ds(k * _LANE, _LANE)]
        v = jnp.maximum(plsc.cummax(v), carry)
        mbuf[pl.ds(k * _LANE, _LANE)] = v - 1
        return jnp.max(v)

    lax.fori_loop(0, _C // _LANE, ph4, jnp.int32(0))

    pltpu.sync_copy(mbuf, anc_hbm.at[pl.ds(s, _C)])

    @pl.loop(0, _C, step=_CHK)
    def _(off):
        @pl.loop(0, _CHK, step=_GWIN)
        def _(c):
            pltpu.async_copy(
                parts_hbm.at[mbuf.at[pl.ds(off + c, _GWIN)]],
                rowsbuf.at[pl.ds(c, _GWIN)], gsem)

        @pl.loop(0, _CHK, step=_GWIN)
        def _(c):
            pltpu.make_async_copy(
                parts_hbm.at[mbuf.at[pl.ds(off + c, _GWIN)]],
                rowsbuf.at[pl.ds(c, _GWIN)], gsem).wait()

        pltpu.sync_copy(rowsbuf, rows_hbm.at[pl.ds(s + off, _CHK)])


def _sc_resample(cum, cs, scal, particles):
    mesh = plsc.VectorSubcoreMesh(core_axis_name="c", subcore_axis_name="s")
    k = pl.kernel(
        _sc_resample_kernel,
        out_type=(
            jax.ShapeDtypeStruct((N,), jnp.int32),
            jax.ShapeDtypeStruct((N, D), jnp.float32),
        ),
        mesh=mesh,
        scratch_types=[
            pltpu.VMEM((_CS,), jnp.float32),
            pltpu.VMEM((_CHK,), jnp.float32),
            pltpu.VMEM((_C,), jnp.int32),
            pltpu.VMEM((_CHK, D), jnp.float32),
            pltpu.VMEM((2, _LANE), jnp.float32),
            pltpu.SemaphoreType.DMA,
        ],
        compiler_params=pltpu.CompilerParams(use_tc_tiling_on_sc=False,
                                             needs_layout_passes=False),
    )
    return k(cum, cs, scal, particles)


_RB = 8192
_NBLK = N // _RB


def _tc_math_kernel(pr_ref, noise_ref, logw_ref, at_ref, ct_ref, obs_ref,
                    lsx_ref, lsy_ref, scal_ref,
                    lw_out, next_out, inc_out):
    x = pr_ref[...]
    noise = noise_ref[...]
    a_t = at_ref[...]
    c_t = ct_ref[...]
    obs = obs_ref[...]
    lsx = lsx_ref[...]
    lsy = lsy_ref[...]
    flag = scal_ref[0, 0]

    mu = jnp.dot(x, a_t, preferred_element_type=jnp.float32)
    sig_p_log = lsx + jnp.float32(np.log(1.1))
    nxt = mu + noise * jnp.exp(sig_p_log)
    d = nxt - mu

    inv_sx = jnp.exp(-lsx)
    inv_sp = jnp.exp(-sig_p_log)
    trans = jnp.sum(-0.5 * (d * inv_sx) ** 2 - lsx - 0.5 * LOG2PI, axis=-1)
    prop = jnp.sum(-0.5 * (d * inv_sp) ** 2 - sig_p_log - 0.5 * LOG2PI,
                   axis=-1)
    emis_mu = jnp.dot(nxt, c_t, preferred_element_type=jnp.float32)
    e = (obs - emis_mu) * jnp.exp(-lsy)
    emis = jnp.sum(-0.5 * e * e - lsy - 0.5 * LOG2PI, axis=-1)
    inc = trans + emis - prop

    lw_r = flag * jnp.float32(-np.log(float(N))) + (1.0 - flag) * logw_ref[0, 0, :]
    lw_out[0, 0, :] = lw_r + inc
    inc_out[0, 0, :] = inc
    next_out[...] = nxt


def _tc_math(particles_r, noise, log_w, A, C, log_sigma_x, log_sigma_y,
             observation, flag):
    logw3 = log_w.reshape(_NBLK, 1, _RB)
    row_spec = pl.BlockSpec((_RB, D), lambda i: (i, 0))
    vec_spec = pl.BlockSpec((1, 1, _RB), lambda i: (i, 0, 0))
    full2 = pl.BlockSpec((D, D), lambda i: (0, 0))
    full1 = pl.BlockSpec((1, D), lambda i: (0, 0))
    out = pl.pallas_call(
        _tc_math_kernel,
        grid=(_NBLK,),
        in_specs=[row_spec, row_spec, vec_spec, full2, full2, full1, full1,
                  full1, pl.BlockSpec((1, 1), lambda i: (0, 0))],
        out_specs=[vec_spec, row_spec, vec_spec],
        out_shape=[
            jax.ShapeDtypeStruct((_NBLK, 1, _RB), jnp.float32),
            jax.ShapeDtypeStruct((N, D), jnp.float32),
            jax.ShapeDtypeStruct((_NBLK, 1, _RB), jnp.float32),
        ],
    )(particles_r, noise, logw3, A.T, C.T, observation.reshape(1, D),
      log_sigma_x.reshape(1, D), log_sigma_y.reshape(1, D),
      flag.reshape(1, 1))
    return out[0].reshape(N), out[1], out[2].reshape(N)


def kernel(log_w, particles, observation, A, C, log_sigma_x, log_sigma_y):
    n = log_w.shape[0]
    step_key = jax.random.key(42)
    resample_key, proposal_key = jax.random.split(step_key)

    w = jax.nn.softmax(log_w)
    ess_e = 1.0 / (jnp.sum(w * w) * n)
    cum = jnp.cumsum(w)
    u0 = jax.random.uniform(resample_key, ())
    do_resample = ess_e < 0.5

    noise = jax.random.normal(proposal_key, particles.shape, particles.dtype)

    flag = do_resample.astype(jnp.float32)
    cs = cum.reshape(_CS, N // _CS)[:, -1]
    scal = jnp.stack([jnp.full((_LANE,), u0, jnp.float32),
                      jnp.full((_LANE,), flag, jnp.float32)])
    ancestor_ix, particles_r = _sc_resample(cum, cs, scal, particles)

    log_w_new, next_particles, inc_weight = _tc_math(
        particles_r, noise, log_w, A, C, log_sigma_x, log_sigma_y,
        observation, flag)

    return (log_w_new, next_particles, ess_e, ancestor_ix, inc_weight)


# speedup vs baseline: 34.7957x; 2.8853x over previous
import functools

import jax
import jax.numpy as jnp
import numpy as np
from jax import lax
from jax.experimental import pallas as pl
from jax.experimental.pallas import tpu as pltpu
from jax.experimental.pallas import tpu_sc as plsc

N = 1048576
D = 16
LOG2PI = float(np.log(2.0 * np.pi))

_NC, _NS = 2, 16
_NW = _NC * _NS
_C = N // _NW
_CS = 1024
_CHK = 2048
_GWIN = 128
_LANE = 16


def _lanes():
    return lax.iota(jnp.int32, _LANE)


def _a_of(cumv, jv, u0v, flagv):
    T = cumv * jnp.float32(N)
    y = T - u0v
    i0 = y.astype(jnp.int32)
    base = i0 - 2
    cnt = jnp.zeros((_LANE,), jnp.int32)
    for m in range(6):
        k = base + m
        cond = ((u0v + k.astype(jnp.float32)) <= T) | (k < 0)
        cnt = cnt + cond.astype(jnp.int32)
    a = jnp.clip(base + cnt, 0, N)
    return jnp.where(flagv, a, jv + 1)


def _shift1(v, carry):
    lanes = _lanes()
    rot = v.at[(lanes + (_LANE - 1)) & (_LANE - 1)].get(
        mode="promise_in_bounds")
    return jnp.where(lanes == 0, carry, rot)


def _sc_resample_kernel(cum_hbm, cs_hbm, scal_hbm, parts_hbm,
                        anc_hbm, rows_hbm,
                        csbuf, cumbuf, mbuf, rowsbuf, scalbuf, gsem):
    wid = lax.axis_index("s") * _NC + lax.axis_index("c")
    s = wid * _C

    pltpu.sync_copy(cs_hbm, csbuf)
    pltpu.sync_copy(scal_hbm, scalbuf)
    u0v = scalbuf[0, :]
    flagv = scalbuf[1, :] > jnp.float32(0.5)

    def ph1(k, st):
        cnt, smax = st
        cv = csbuf[pl.ds(k * _LANE, _LANE)]
        jv = (_lanes() + k * _LANE) * _CS + (_CS - 1)
        sv = _a_of(cv, jv, u0v, flagv)
        le = sv <= s
        cnt = cnt + jnp.sum(le.astype(jnp.int32))
        smax = jnp.maximum(smax, jnp.max(jnp.where(le, sv, 0)))
        return cnt, smax

    kk, carry0 = lax.fori_loop(0, _CS // _LANE, ph1,
                               (jnp.int32(0), jnp.int32(0)))
    t0 = kk * _CS

    zer = jnp.zeros((_LANE,), jnp.int32)

    @pl.loop(0, _C, step=_LANE)
    def _(i):
        mbuf[pl.ds(i, _LANE)] = zer

    def cond(st):
        return jnp.logical_not(st[2])

    def body(st):
        t, carry_a, _ = st
        tt = pl.multiple_of(jnp.minimum(t, N - _CHK), 8)
        pltpu.sync_copy(cum_hbm.at[pl.ds(tt, _CHK)], cumbuf)

        def inner(k, carry_a):
            cv = cumbuf[pl.ds(k * _LANE, _LANE)]
            jv = _lanes() + (tt + k * _LANE)
            av = _a_of(cv, jv, u0v, flagv)
            aprev = _shift1(av, carry_a)
            pos = jnp.maximum(aprev - s, 0)
            mask = (av > aprev) & (av > s) & (pos < _C)
            plsc.store_scatter(mbuf, [pos], jv + 1, mask=mask)
            return jnp.max(av)

        carry_a = lax.fori_loop(0, _CHK // _LANE, inner, carry_a)
        t2 = tt + _CHK
        return t2, carry_a, (carry_a >= s + _C) | (t2 >= N)

    lax.while_loop(cond, body, (t0, carry0, t0 >= N))

    pltpu.sync_copy(cum_hbm.at[pl.ds(N - _LANE, _LANE)],
                    cumbuf.at[pl.ds(0, _LANE)])
    cv = cumbuf[pl.ds(0, _LANE)]
    av = _a_of(cv, _lanes() + (N - _LANE), u0v, flagv)
    a_last = jnp.max(av)
    pos_t = jnp.maximum(a_last - s, 0)
    maskt = (_lanes() == 0) & ((a_last - s) < _C) & (a_last <= N - 1)
    plsc.store_scatter(mbuf, [jnp.zeros((_LANE,), jnp.int32) + pos_t],
                       jnp.zeros((_LANE,), jnp.int32) + N, mask=maskt)

    def ph4(k, carry):
        v = mbuf[pl.ds(k * _LANE, _LANE)]
        v = jnp.maximum(plsc.cummax(v), carry)
        mbuf[pl.ds(k * _LANE, _LANE)] = v - 1
        return jnp.max(v)

    lax.fori_loop(0, _C // _LANE, ph4, jnp.int32(0))

    pltpu.sync_copy(mbuf, anc_hbm.at[pl.ds(s, _C)])

    @pl.loop(0, _C, step=_CHK)
    def _(off):
        @pl.loop(0, _CHK, step=_GWIN)
        def _(c):
            pltpu.async_copy(
                parts_hbm.at[mbuf.at[pl.ds(off + c, _GWIN)]],
                rowsbuf.at[pl.ds(c, _GWIN)], gsem)

        @pl.loop(0, _CHK, step=_GWIN)
        def _(c):
            pltpu.make_async_copy(
                parts_hbm.at[mbuf.at[pl.ds(off + c, _GWIN)]],
                rowsbuf.at[pl.ds(c, _GWIN)], gsem).wait()

        pltpu.sync_copy(rowsbuf, rows_hbm.at[pl.ds(s + off, _CHK)])


def _sc_resample(cum, cs, scal, particles):
    mesh = plsc.VectorSubcoreMesh(core_axis_name="c", subcore_axis_name="s")
    k = pl.kernel(
        _sc_resample_kernel,
        out_type=(
            jax.ShapeDtypeStruct((N,), jnp.int32),
            jax.ShapeDtypeStruct((N, D), jnp.float32),
        ),
        mesh=mesh,
        scratch_types=[
            pltpu.VMEM((_CS,), jnp.float32),
            pltpu.VMEM((_CHK,), jnp.float32),
            pltpu.VMEM((_C,), jnp.int32),
            pltpu.VMEM((_CHK, D), jnp.float32),
            pltpu.VMEM((2, _LANE), jnp.float32),
            pltpu.SemaphoreType.DMA,
        ],
        compiler_params=pltpu.CompilerParams(use_tc_tiling_on_sc=False,
                                             needs_layout_passes=False),
    )
    return k(cum, cs, scal, particles)


_P = N * D // 128
_BP = 1024
_NPB = _P // _BP
_BN = 512
_NNB = _P // _BN

_ROT = ((13, 15, 26, 6), (17, 29, 16, 24))


def _tc_noise_kernel(ks_ref, out_ref):
    ks0 = ks_ref[0, 0]
    ks1 = ks_ref[0, 1]
    ks2 = ks0 ^ ks1 ^ jnp.uint32(0x1BD11BDA)
    i = pl.program_id(0)
    bi = lax.broadcasted_iota(jnp.uint32, (_BN, 128), 0)
    li = lax.broadcasted_iota(jnp.uint32, (_BN, 128), 1)
    e = (bi + jnp.uint32(i * _BN).astype(jnp.uint32)) * jnp.uint32(128) + li

    x0 = jnp.zeros((_BN, 128), jnp.uint32) + ks0
    x1 = e + ks1
    inj = ((ks1, ks2, 1), (ks2, ks0, 2), (ks0, ks1, 3), (ks1, ks2, 4),
           (ks2, ks0, 5))
    for g in range(5):
        for r in _ROT[g % 2]:
            x0 = x0 + x1
            x1 = (x1 << jnp.uint32(r)) | (x1 >> jnp.uint32(32 - r))
            x1 = x1 ^ x0
        a, b, c = inj[g]
        x0 = x0 + a
        x1 = x1 + b + jnp.uint32(c)
    bits = x0 ^ x1

    flo = lax.bitcast_convert_type((bits >> jnp.uint32(9))
                                   | jnp.uint32(0x3F800000), jnp.float32)
    f = flo - jnp.float32(1.0)
    lo = jnp.float32(np.nextafter(np.float32(-1.0), np.float32(0.0)))
    u = jnp.maximum(lo, f * (jnp.float32(1.0) - lo) + lo)

    w = -jnp.log((jnp.float32(1.0) - u) * (jnp.float32(1.0) + u))
    p_small = (2.81022636e-08, 3.43273939e-07, -3.5233877e-06,
               -4.39150654e-06, 0.00021858087, -0.00125372503,
               -0.00417768164, 0.246640727, 1.50140941)
    p_big = (-0.000200214257, 0.000100950558, 0.00134934322, -0.00367342844,
             0.00573950773, -0.0076224613, 0.00943887047, 1.00167406,
             2.83297682)
    ws = w - jnp.float32(2.5)
    wb = jnp.sqrt(w) - jnp.float32(3.0)
    ps = jnp.full((_BN, 128), p_small[0], jnp.float32)
    pb = jnp.full((_BN, 128), p_big[0], jnp.float32)
    for cs, cb in zip(p_small[1:], p_big[1:]):
        ps = ps * ws + jnp.float32(cs)
        pb = pb * wb + jnp.float32(cb)
    p = jnp.where(w < jnp.float32(5.0), ps, pb)
    out_ref[...] = jnp.float32(np.sqrt(2.0)) * (p * u)


def _tc_noise(ks):
    return pl.pallas_call(
        _tc_noise_kernel,
        grid=(_NNB,),
        in_specs=[pl.BlockSpec((1, 2), lambda i: (0, 0))],
        out_specs=pl.BlockSpec((_BN, 128), lambda i: (i, 0)),
        out_shape=jax.ShapeDtypeStruct((_P, 128), jnp.float32),
    )(ks)


def _tc_math_kernel(pr_ref, nz_ref, lwp_ref, bda_ref, bdc_ref, seg_ref,
                    obs_ref, lsx_ref, lsy_ref, scal_ref,
                    lw_out, next_out, inc_out):
    x = pr_ref[...]
    noise = nz_ref[...]
    bda = bda_ref[...]
    bdc = bdc_ref[...]
    seg = seg_ref[...]
    obs = obs_ref[...]
    lsx = lsx_ref[...]
    lsy = lsy_ref[...]
    flag = scal_ref[0, 0]

    mu = jnp.dot(x, bda, preferred_element_type=jnp.float32)
    sig_p_log = lsx + jnp.float32(np.log(1.1))
    nxt = mu + noise * jnp.exp(sig_p_log)
    d = nxt - mu

    inv_sx = jnp.exp(-lsx)
    inv_sp = jnp.exp(-sig_p_log)
    t = (-0.5 * (d * inv_sx) ** 2 - lsx) - (-0.5 * (d * inv_sp) ** 2
                                            - sig_p_log)
    emis_mu = jnp.dot(nxt, bdc, preferred_element_type=jnp.float32)
    ee = (obs - emis_mu) * jnp.exp(-lsy)
    t = t + (-0.5 * ee * ee - lsy - jnp.float32(0.5 * LOG2PI))
    inc = jnp.dot(t, seg, preferred_element_type=jnp.float32)

    lw_r = flag * jnp.float32(-np.log(float(N))) \
        + (jnp.float32(1.0) - flag) * lwp_ref[...]
    lw_out[...] = lw_r + inc
    inc_out[...] = inc
    next_out[...] = nxt


def _tc_math(pr_p, noise_p, log_w, A, C, log_sigma_x, log_sigma_y,
             observation, flag):
    lwp = log_w.reshape(_P, 8)
    bda = jnp.kron(jnp.eye(8, dtype=jnp.float32), A.T)
    bdc = jnp.kron(jnp.eye(8, dtype=jnp.float32), C.T)
    seg = jnp.kron(jnp.eye(8, dtype=jnp.float32),
                   jnp.ones((D, 1), jnp.float32))
    obs_t = jnp.tile(observation, 8).reshape(1, 128)
    lsx_t = jnp.tile(log_sigma_x, 8).reshape(1, 128)
    lsy_t = jnp.tile(log_sigma_y, 8).reshape(1, 128)

    big = pl.BlockSpec((_BP, 128), lambda i: (i, 0))
    sml = pl.BlockSpec((_BP, 8), lambda i: (i, 0))
    out = pl.pallas_call(
        _tc_math_kernel,
        grid=(_NPB,),
        in_specs=[big, big, sml,
                  pl.BlockSpec((128, 128), lambda i: (0, 0)),
                  pl.BlockSpec((128, 128), lambda i: (0, 0)),
                  pl.BlockSpec((128, 8), lambda i: (0, 0)),
                  pl.BlockSpec((1, 128), lambda i: (0, 0)),
                  pl.BlockSpec((1, 128), lambda i: (0, 0)),
                  pl.BlockSpec((1, 128), lambda i: (0, 0)),
                  pl.BlockSpec((1, 1), lambda i: (0, 0))],
        out_specs=[sml, big, sml],
        out_shape=[
            jax.ShapeDtypeStruct((_P, 8), jnp.float32),
            jax.ShapeDtypeStruct((_P, 128), jnp.float32),
            jax.ShapeDtypeStruct((_P, 8), jnp.float32),
        ],
    )(pr_p, noise_p, lwp, bda, bdc, seg, obs_t, lsx_t, lsy_t,
      flag.reshape(1, 1))
    return (out[0].reshape(N), out[1].reshape(N, D), out[2].reshape(N))


def kernel(log_w, particles, observation, A, C, log_sigma_x, log_sigma_y):
    n = log_w.shape[0]
    step_key = jax.random.key(42)
    resample_key, proposal_key = jax.random.split(step_key)

    w = jax.nn.softmax(log_w)
    ess_e = 1.0 / (jnp.sum(w * w) * n)
    cum = jnp.cumsum(w)
    u0 = jax.random.uniform(resample_key, ())
    do_resample = ess_e < 0.5

    flag = do_resample.astype(jnp.float32)
    cs = cum.reshape(_CS, N // _CS)[:, -1]
    scal = jnp.stack([jnp.full((_LANE,), u0, jnp.float32),
                      jnp.full((_LANE,), flag, jnp.float32)])
    ks = jax.random.key_data(proposal_key).astype(jnp.uint32).reshape(1, 2)
    noise_p = _tc_noise(ks)
    ancestor_ix, particles_r = _sc_resample(cum, cs, scal, particles)

    log_w_new, next_particles, inc_weight = _tc_math(
        particles_r.reshape(_P, 128), noise_p, log_w, A, C,
        log_sigma_x, log_sigma_y, observation, flag)

    return (log_w_new, next_particles, ess_e, ancestor_ix, inc_weight)
